# Initial kernel scaffold; baseline (speedup 1.0000x reference)
import jax
import jax.numpy as jnp
from jax.experimental import pallas as pl


def kernel(logits_1, logits_2, targets, epoch):
    raise NotImplementedError("write your pallas kernel here")


# speedup vs baseline: 2.7188x; 2.7188x over previous
import numpy as np
import jax
import jax.numpy as jnp
from jax import lax
from jax.experimental import pallas as pl
from jax.experimental.pallas import tpu as pltpu

_B = 4096
_C = 1000
_RB = 1024
_NBLK = _B // _RB
_GR = _B // 128

_FORGET = 0.2
_GRADUAL = 10
_EPOCHS = 100


def _forget_schedule():
    rs = np.ones(_EPOCHS) * _FORGET
    rs[:_GRADUAL] = np.linspace(0, _FORGET, _GRADUAL)
    return rs


def _ce_block(x, tgt):
    m = jnp.max(x, axis=1, keepdims=True)
    m = jnp.where(jnp.isfinite(m), m, 0.0)
    lse = jnp.log(jnp.sum(jnp.exp(x - m), axis=1, keepdims=True)) + m
    col = lax.broadcasted_iota(jnp.int32, x.shape, 1)
    tv = jnp.sum(jnp.where(col == tgt[:, None], x, 0.0), axis=1, keepdims=True)
    return lse - tv


def _order_key(x):
    u = lax.bitcast_convert_type(x, jnp.uint32)
    neg = u >= jnp.uint32(0x80000000)
    return jnp.where(neg, ~u, u | jnp.uint32(0x80000000))


def _kth_thresholds2(u1, u2, idx, k):
    def vstep(_, st):
        lo1, hi1, lo2, hi2 = st
        m1 = lo1 + ((hi1 - lo1) >> jnp.uint32(1))
        m2 = lo2 + ((hi2 - lo2) >> jnp.uint32(1))
        c1 = jnp.sum((u1 <= m1).astype(jnp.int32))
        c2 = jnp.sum((u2 <= m2).astype(jnp.int32))
        return (jnp.where(c1 >= k, lo1, m1 + jnp.uint32(1)),
                jnp.where(c1 >= k, m1, hi1),
                jnp.where(c2 >= k, lo2, m2 + jnp.uint32(1)),
                jnp.where(c2 >= k, m2, hi2))

    z, f = jnp.uint32(0), jnp.uint32(0xFFFFFFFF)
    _, v1, _, v2 = lax.fori_loop(0, 32, vstep, (z, f, z, f))
    nl1 = jnp.sum((u1 < v1).astype(jnp.int32))
    nl2 = jnp.sum((u2 < v2).astype(jnp.int32))
    eq1 = u1 == v1
    eq2 = u2 == v2

    def tstep(_, st):
        lo1, hi1, lo2, hi2 = st
        m1 = lo1 + ((hi1 - lo1) >> jnp.int32(1))
        m2 = lo2 + ((hi2 - lo2) >> jnp.int32(1))
        c1 = nl1 + jnp.sum((eq1 & (idx < m1)).astype(jnp.int32))
        c2 = nl2 + jnp.sum((eq2 & (idx < m2)).astype(jnp.int32))
        return (jnp.where(c1 >= k, lo1, m1 + 1), jnp.where(c1 >= k, m1, hi1),
                jnp.where(c2 >= k, lo2, m2 + 1), jnp.where(c2 >= k, m2, hi2))

    zi, bi = jnp.int32(0), jnp.int32(_B)
    _, t1, _, t2 = lax.fori_loop(0, 13, tstep, (zi, bi, zi, bi))
    return v1, t1, v2, t2


def _tc_body(x1_ref, x2_ref, tgt_ref, k_ref, o1_ref, o2_ref, l1_s, l2_s):
    g = pl.program_id(0)
    gr = _RB // 128
    tgt = tgt_ref[...]
    l1_s[pl.ds(g * gr, gr), :] = _ce_block(x1_ref[...], tgt).reshape(gr, 128)
    l2_s[pl.ds(g * gr, gr), :] = _ce_block(x2_ref[...], tgt).reshape(gr, 128)

    @pl.when(g == _NBLK - 1)
    def _():
        k = k_ref[0]
        l1 = l1_s[...]
        l2 = l2_s[...]
        u1 = _order_key(l1)
        u2 = _order_key(l2)
        idx = (lax.broadcasted_iota(jnp.int32, (_GR, 128), 0) * 128
               + lax.broadcasted_iota(jnp.int32, (_GR, 128), 1))
        v1, t1, v2, t2 = _kth_thresholds2(u1, u2, idx, k)
        sel2 = (u2 < v2) | ((u2 == v2) & (idx < t2))
        sel1 = (u1 < v1) | ((u1 == v1) & (idx < t1))
        den = k.astype(jnp.float32)
        o1_ref[0] = jnp.sum(jnp.where(sel2, l1, 0.0)) / den
        o2_ref[0] = jnp.sum(jnp.where(sel1, l2, 0.0)) / den


def kernel(logits_1, logits_2, targets, epoch):
    rs = jnp.asarray(_forget_schedule())
    remember_rate = 1.0 - rs[epoch]
    num_remember = (remember_rate * logits_1.shape[0]).astype(jnp.int32)
    k_arr = num_remember.reshape(1)
    out1, out2 = pl.pallas_call(
        _tc_body,
        grid=(_NBLK,),
        in_specs=[
            pl.BlockSpec((_RB, _C), lambda g: (g, 0)),
            pl.BlockSpec((_RB, _C), lambda g: (g, 0)),
            pl.BlockSpec((_RB,), lambda g: (g,)),
            pl.BlockSpec(memory_space=pltpu.SMEM),
        ],
        out_specs=[
            pl.BlockSpec(memory_space=pltpu.SMEM),
            pl.BlockSpec(memory_space=pltpu.SMEM),
        ],
        out_shape=[jax.ShapeDtypeStruct((1,), jnp.float32)] * 2,
        scratch_shapes=[pltpu.VMEM((_GR, 128), jnp.float32)] * 2,
    )(logits_1, logits_2, targets.astype(jnp.int32), k_arr)
    return (out1[0], out2[0])
